# baseline (speedup 1.0000x reference)
import functools

import jax
import jax.numpy as jnp
from jax import lax
from jax.experimental import pallas as pl
from jax.experimental.pallas import tpu as pltpu
from jax.experimental.pallas import tpu_sc as plsc

BATCH = 16384
NUM_FIELDS = 26
EMBED_DIM = 16
NUM_EMB = 2600000
TOTAL = BATCH * NUM_FIELDS

_info = plsc.get_sparse_core_info()
_NC, _NS = _info.num_cores, _info.num_subcores
_NW = _NC * _NS
_BATCH_PER_W = BATCH // _NW
_CB = 4
_CHUNK = _CB * NUM_FIELDS
_NSTEP = _BATCH_PER_W // _CB

_mesh = plsc.VectorSubcoreMesh(core_axis_name="c", subcore_axis_name="s")


@functools.partial(
    pl.kernel,
    mesh=_mesh,
    out_type=jax.ShapeDtypeStruct((BATCH, NUM_FIELDS, EMBED_DIM), jnp.float32),
    scratch_types=[
        pltpu.VMEM((112,), jnp.int32),
        pltpu.SemaphoreType.DMA,
        pltpu.SemaphoreType.DMA,
    ],
    compiler_params=pltpu.CompilerParams(
        use_tc_tiling_on_sc=True, needs_layout_passes=False),
)
def _embed_gather(table_hbm, idx_hbm, out_hbm, x_v, isem, rsem):
    wid = lax.axis_index("s") * _NC + lax.axis_index("c")
    lanes = lax.iota(jnp.int32, 16)

    def step(i, carry):
        b0 = wid * _BATCH_PER_W + i * _CB
        r0 = b0 * NUM_FIELDS
        pltpu.async_copy(idx_hbm.at[pl.ds(r0, _CHUNK)],
                         x_v.at[pl.ds(0, _CHUNK)], isem).wait()
        copies = []
        for r in range(_CHUNK):
            if r % 16 == 0:
                vblk = x_v[pl.ds(r, 16)]
            v = jnp.sum(jnp.where(lanes == (r % 16), vblk, 0))
            g = lax.shift_right_logical(v, 3)
            s = jnp.bitwise_and(v, 7)
            b_loc, f = divmod(r, NUM_FIELDS)
            copies.append(pltpu.async_copy(
                table_hbm.at[g, s], out_hbm.at[b0 + b_loc, f], rsem))
        for cp in copies:
            cp.wait()
        return carry

    lax.fori_loop(0, _NSTEP, step, 0)


def kernel(table, x):
    table3 = table.reshape(NUM_EMB // 8, 8, EMBED_DIM)
    flat = x.reshape(TOTAL)
    return _embed_gather(table3, flat)


# speedup vs baseline: 5.1825x; 5.1825x over previous
import functools

import jax
import jax.numpy as jnp
from jax import lax
from jax.experimental import pallas as pl
from jax.experimental.pallas import tpu as pltpu
from jax.experimental.pallas import tpu_sc as plsc

BATCH = 16384
NUM_FIELDS = 26
EMBED_DIM = 16
TOTAL = BATCH * NUM_FIELDS

_info = plsc.get_sparse_core_info()
_NC, _NS = _info.num_cores, _info.num_subcores
_NW = _NC * _NS
_BATCH_PER_W = BATCH // _NW
_CB = 8
_CHUNK = _CB * NUM_FIELDS
_NSTEP = _BATCH_PER_W // _CB

_mesh = plsc.VectorSubcoreMesh(core_axis_name="c", subcore_axis_name="s")


@functools.partial(
    pl.kernel,
    mesh=_mesh,
    out_type=jax.ShapeDtypeStruct((BATCH, 32, 128), jnp.float32),
    scratch_types=[
        pltpu.VMEM((_CHUNK,), jnp.int32),
        pltpu.VMEM((_CHUNK, EMBED_DIM), jnp.float32),
        pltpu.SemaphoreType.DMA,
    ],
    compiler_params=pltpu.CompilerParams(use_tc_tiling_on_sc=False),
)
def _gather_rows(table_hbm, idx_hbm, out_hbm, idx_v, rows_v, sem):
    wid = lax.axis_index("s") * _NC + lax.axis_index("c")

    def step(i, carry):
        b0 = wid * _BATCH_PER_W + i * _CB
        r0 = b0 * NUM_FIELDS
        pltpu.sync_copy(idx_hbm.at[pl.ds(r0, _CHUNK)], idx_v)
        pltpu.async_copy(table_hbm.at[idx_v], rows_v, sem).wait()
        copies = [
            pltpu.async_copy(
                rows_v.at[pl.ds(cb * NUM_FIELDS, NUM_FIELDS), :],
                out_hbm.at[b0 + cb, pl.ds(0, NUM_FIELDS),
                           pl.ds(0, EMBED_DIM)],
                sem)
            for cb in range(_CB)
        ]
        for cp in copies:
            cp.wait()
        return carry

    lax.fori_loop(0, _NSTEP, step, 0)


def kernel(table, x):
    flat = x.reshape(TOTAL)
    out_pad = _gather_rows(table, flat)
    return out_pad[:, :NUM_FIELDS, :EMBED_DIM]
